# Initial kernel scaffold; baseline (speedup 1.0000x reference)
import jax
import jax.numpy as jnp
from jax.experimental import pallas as pl


def kernel(hidden_states, gate_weight, w_gate, w_up, w_down, sw_gate, sw_up, sw_down):
    raise NotImplementedError("write your pallas kernel here")


# speedup vs baseline: 1.1004x; 1.1004x over previous
import functools

import jax
import jax.numpy as jnp
from jax.experimental import pallas as pl
from jax.experimental.pallas import tpu as pltpu

E = 8
EP = 128
TOP_K = 2


def _router_body(x_ref, gwt_ref, comb_ref, xb_ref):
    t = x_ref.shape[0]
    x = x_ref[...]
    logits = jax.lax.dot_general(
        x.astype(jnp.bfloat16), gwt_ref[...].astype(jnp.bfloat16),
        (((1,), (0,)), ((), ())),
        preferred_element_type=jnp.float32)
    col = jax.lax.broadcasted_iota(jnp.int32, (t, EP), 1)
    logits = jnp.where(col < E, logits, jnp.float32(-1e30))
    m = jnp.max(logits, axis=1, keepdims=True)
    ex = jnp.exp(logits - m)
    scores = ex / jnp.sum(ex, axis=1, keepdims=True)
    s1 = jnp.max(scores, axis=1, keepdims=True)
    i1 = jnp.min(jnp.where(scores == s1, col, EP), axis=1, keepdims=True)
    oh1 = col == i1
    s2 = jnp.max(jnp.where(oh1, jnp.float32(-1.0), scores), axis=1,
                 keepdims=True)
    i2 = jnp.min(jnp.where((scores == s2) & (~oh1), col, EP), axis=1,
                 keepdims=True)
    oh2 = col == i2
    comb_ref[...] = jnp.where(oh1 | oh2, scores, jnp.float32(0.0))
    xb_ref[...] = x.astype(jnp.bfloat16)


def _dense_expert_body(comb_ref, xb_ref, wg_ref, wu_ref, wd_ref, out_ref):
    e = pl.program_id(0)
    f = pl.program_id(1)

    @pl.when((e == 0) & (f == 0))
    def _():
        out_ref[...] = jnp.zeros_like(out_ref)

    xb = xb_ref[...]
    wg = wg_ref[0].astype(jnp.bfloat16)
    wu = wu_ref[0].astype(jnp.bfloat16)
    wd = wd_ref[0].astype(jnp.bfloat16)
    nt = (((1,), (1,)), ((), ()))
    a = jax.lax.dot_general(xb, wg, nt, preferred_element_type=jnp.float32)
    b = jax.lax.dot_general(xb, wu, nt, preferred_element_type=jnp.float32)
    h = (a * jax.nn.sigmoid(a) * b).astype(jnp.bfloat16)
    y = jax.lax.dot_general(h, wd, nt, preferred_element_type=jnp.float32)
    basis = (jax.lax.broadcasted_iota(jnp.int32, (EP, 1), 0) == e
             ).astype(jnp.float32)
    comb_col = jax.lax.dot_general(
        comb_ref[...], basis, (((1,), (0,)), ((), ())),
        precision=jax.lax.Precision.HIGHEST,
        preferred_element_type=jnp.float32)
    out_ref[...] += y * comb_col


def _shared_expert_body(xb_ref, base_ref, wg_ref, wu_ref, wd_ref, out_ref):
    f = pl.program_id(0)

    @pl.when(f == 0)
    def _():
        out_ref[...] = base_ref[...]

    xb = xb_ref[...]
    wg = wg_ref[...].astype(jnp.bfloat16)
    wu = wu_ref[...].astype(jnp.bfloat16)
    wd = wd_ref[...].astype(jnp.bfloat16)
    nt = (((1,), (1,)), ((), ()))
    a = jax.lax.dot_general(xb, wg, nt, preferred_element_type=jnp.float32)
    b = jax.lax.dot_general(xb, wu, nt, preferred_element_type=jnp.float32)
    h = (a * jax.nn.sigmoid(a) * b).astype(jnp.bfloat16)
    out_ref[...] += jax.lax.dot_general(
        h, wd, nt, preferred_element_type=jnp.float32)


def kernel(hidden_states, gate_weight, w_gate, w_up, w_down,
           sw_gate, sw_up, sw_down):
    b, s, d = hidden_states.shape
    t = b * s
    e, ff, _ = w_gate.shape
    sff = sw_gate.shape[0]
    x = hidden_states.reshape(t, d)
    gwt = jnp.pad(gate_weight.T, ((0, 0), (0, EP - e)))

    comb, xb = pl.pallas_call(
        _router_body,
        out_shape=(
            jax.ShapeDtypeStruct((t, EP), jnp.float32),
            jax.ShapeDtypeStruct((t, d), jnp.bfloat16),
        ),
    )(x, gwt)

    tf = min(512, ff)
    nf = ff // tf
    routed = pl.pallas_call(
        _dense_expert_body,
        grid=(e, nf),
        in_specs=[
            pl.BlockSpec((t, EP), lambda ei, fi: (0, 0)),
            pl.BlockSpec((t, d), lambda ei, fi: (0, 0)),
            pl.BlockSpec((1, tf, d), lambda ei, fi: (ei, fi, 0)),
            pl.BlockSpec((1, tf, d), lambda ei, fi: (ei, fi, 0)),
            pl.BlockSpec((1, d, tf), lambda ei, fi: (ei, 0, fi)),
        ],
        out_specs=pl.BlockSpec((t, d), lambda ei, fi: (0, 0)),
        out_shape=jax.ShapeDtypeStruct((t, d), jnp.float32),
    )(comb, xb, w_gate, w_up, w_down)

    tfs = min(256, sff)
    nfs = sff // tfs
    out = pl.pallas_call(
        _shared_expert_body,
        grid=(nfs,),
        in_specs=[
            pl.BlockSpec((t, d), lambda fi: (0, 0)),
            pl.BlockSpec((t, d), lambda fi: (0, 0)),
            pl.BlockSpec((tfs, d), lambda fi: (fi, 0)),
            pl.BlockSpec((tfs, d), lambda fi: (fi, 0)),
            pl.BlockSpec((d, tfs), lambda fi: (0, fi)),
        ],
        out_specs=pl.BlockSpec((t, d), lambda fi: (0, 0)),
        out_shape=jax.ShapeDtypeStruct((t, d), jnp.float32),
    )(xb, routed, sw_gate, sw_up, sw_down)

    return out.reshape(b, s, d)


# speedup vs baseline: 1.7702x; 1.6087x over previous
import functools

import jax
import jax.numpy as jnp
from jax import lax
from jax.experimental import pallas as pl
from jax.experimental.pallas import tpu as pltpu
from jax.experimental.pallas import tpu_sc as plsc

E = 8
EP = 128
T = 2048
D = 1024
C = 1024
OVB = 1024
V2 = 3
R = E * C + V2 * OVB
NC, NS = 2, 16
NW = NC * NS
CH = T // NW
TRI = 512


def _router_body(x_ref, gwt_ref, route_ref, cnt_ref, xb_ref,
                 w0b_ref, w1b_ref):
    x = x_ref[...]
    logits = lax.dot_general(
        x.astype(jnp.bfloat16), gwt_ref[...].astype(jnp.bfloat16),
        (((1,), (0,)), ((), ())), preferred_element_type=jnp.float32)
    col = lax.broadcasted_iota(jnp.int32, (T, EP), 1)
    logits = jnp.where(col < E, logits, jnp.float32(-1e30))
    m = jnp.max(logits, axis=1, keepdims=True)
    ex = jnp.exp(logits - m)
    scores = ex / jnp.sum(ex, axis=1, keepdims=True)
    s1 = jnp.max(scores, axis=1, keepdims=True)
    i1 = jnp.min(jnp.where(scores == s1, col, EP), axis=1, keepdims=True)
    oh1 = col == i1
    s2 = jnp.max(jnp.where(oh1, jnp.float32(-1.0), scores), axis=1,
                 keepdims=True)
    i2 = jnp.min(jnp.where((scores == s2) & (~oh1), col, EP), axis=1,
                 keepdims=True)
    oh2 = col == i2

    ri = lax.broadcasted_iota(jnp.int32, (TRI, TRI), 0)
    ci = lax.broadcasted_iota(jnp.int32, (TRI, TRI), 1)
    ltri = (ci < ri).astype(jnp.bfloat16)
    running = jnp.zeros((1, EP), jnp.float32)
    parts = []
    for oh in (oh1, oh2):
        ohf = oh.astype(jnp.float32)
        for blk in range(T // TRI):
            ohb = ohf[blk * TRI:(blk + 1) * TRI, :]
            within = lax.dot_general(
                ltri, ohb.astype(jnp.bfloat16), (((1,), (0,)), ((), ())),
                preferred_element_type=jnp.float32)
            ranks_blk = within + running
            parts.append(jnp.sum(ranks_blk * ohb, axis=1, keepdims=True))
            running = running + jnp.sum(ohb, axis=0, keepdims=True)
    r1 = jnp.concatenate(parts[:4], axis=0)
    r2 = jnp.concatenate(parts[4:], axis=0)
    cnt = running

    ri2 = lax.broadcasted_iota(jnp.int32, (EP, EP), 0)
    ci2 = lax.broadcasted_iota(jnp.int32, (EP, EP), 1)
    mlt = (ri2 < ci2).astype(jnp.float32)
    excl = functools.partial(
        lax.dot_general, dimension_numbers=(((1,), (0,)), ((), ())),
        precision=lax.Precision.HIGHEST, preferred_element_type=jnp.float32)
    off = excl(cnt, mlt)
    ov = jnp.maximum(cnt - C, 0.0)
    ovpad = jnp.floor((ov + (OVB - 1)) / OVB) * OVB
    p2o = excl(ovpad, mlt)

    ohf1 = oh1.astype(jnp.float32)
    ohf2 = oh2.astype(jnp.float32)
    def slot_for(ohf, idx, rk):
        base = jnp.sum(off * ohf, axis=1, keepdims=True)
        p2 = jnp.sum(p2o * ohf, axis=1, keepdims=True)
        ef = idx.astype(jnp.float32)
        return jnp.where(rk < C, ef * C + rk, E * C + p2 + (rk - C))
    slot0 = slot_for(ohf1, i1, r1)
    slot1 = slot_for(ohf2, i2, r2)
    w0 = s1
    w1 = s2

    g = (jnp.where(col == 0, slot0, 0.0) + jnp.where(col == 1, slot1, 0.0)
         + jnp.where(col == 2, w0, 0.0) + jnp.where(col == 3, w1, 0.0))
    route_ref[...] = lax.transpose(g, (1, 0))
    cnt_ref[...] = cnt
    xb_ref[...] = x.astype(jnp.bfloat16)
    w0b_ref[...] = jnp.broadcast_to(w0, (T, 16))
    w1b_ref[...] = jnp.broadcast_to(w1, (T, 16))


def _make_dispatch():
    mesh = plsc.VectorSubcoreMesh(core_axis_name="c", subcore_axis_name="s")

    @functools.partial(
        pl.kernel, mesh=mesh,
        out_type=jax.ShapeDtypeStruct((R, D), jnp.float32),
        scratch_types=[
            pltpu.VMEM((CH, D), jnp.float32),
            pltpu.VMEM((CH,), jnp.float32),
            pltpu.VMEM((CH,), jnp.int32),
            pltpu.SemaphoreType.DMA,
        ],
    )
    def dispatch(x_hbm, route_hbm, xd_hbm, rows_v, sf_v, idx_v, sem):
        wid = lax.axis_index("s") * NC + lax.axis_index("c")
        base = wid * CH
        pltpu.sync_copy(x_hbm.at[pl.ds(base, CH)], rows_v)
        for k in range(2):
            pltpu.sync_copy(route_hbm.at[k, pl.ds(base, CH)], sf_v)
            for c in range(CH // 16):
                idx_v[pl.ds(c * 16, 16)] = (
                    sf_v[pl.ds(c * 16, 16)].astype(jnp.int32))
            pltpu.async_copy(rows_v, xd_hbm.at[idx_v], sem).wait()

    return dispatch


def _pass1_body(m_ref, x_ref, wg_ref, wu_ref, wd_ref, y_ref, xb_scr):
    g = pl.program_id(0)
    f = pl.program_id(1)
    na = m_ref[V2]

    @pl.when(g < E + na)
    def _():
        @pl.when(f == 0)
        def _():
            xb_scr[...] = x_ref[...].astype(jnp.bfloat16)
            y_ref[...] = jnp.zeros_like(y_ref)

        xb = xb_scr[...]
        wg = wg_ref[0].astype(jnp.bfloat16)
        wu = wu_ref[0].astype(jnp.bfloat16)
        wd = wd_ref[0].astype(jnp.bfloat16)
        nt = (((1,), (1,)), ((), ()))
        a = lax.dot_general(xb, wg, nt, preferred_element_type=jnp.float32)
        bb = lax.dot_general(xb, wu, nt, preferred_element_type=jnp.float32)
        h = (a * jax.nn.sigmoid(a) * bb).astype(jnp.bfloat16)
        y_ref[...] += lax.dot_general(h, wd, nt,
                                      preferred_element_type=jnp.float32)


def _shared_body(xb_ref, wg_ref, wu_ref, wd_ref, out_ref):
    f = pl.program_id(0)

    @pl.when(f == 0)
    def _():
        out_ref[...] = jnp.zeros_like(out_ref)

    xb = xb_ref[...]
    wg = wg_ref[...].astype(jnp.bfloat16)
    wu = wu_ref[...].astype(jnp.bfloat16)
    wd = wd_ref[...].astype(jnp.bfloat16)
    nt = (((1,), (1,)), ((), ()))
    a = lax.dot_general(xb, wg, nt, preferred_element_type=jnp.float32)
    b = lax.dot_general(xb, wu, nt, preferred_element_type=jnp.float32)
    h = (a * jax.nn.sigmoid(a) * b).astype(jnp.bfloat16)
    out_ref[...] += lax.dot_general(h, wd, nt,
                                    preferred_element_type=jnp.float32)


def _make_combine():
    mesh = plsc.VectorSubcoreMesh(core_axis_name="c", subcore_axis_name="s")
    chunks = CH // 16

    @functools.partial(
        pl.kernel, mesh=mesh,
        out_type=jax.ShapeDtypeStruct((T, D), jnp.float32),
        scratch_types=[
            pltpu.VMEM((16, D), jnp.float32),
            pltpu.VMEM((16, D), jnp.float32),
            pltpu.VMEM((16, D), jnp.float32),
            pltpu.VMEM((16,), jnp.float32),
            pltpu.VMEM((16,), jnp.int32),
            pltpu.VMEM((16,), jnp.int32),
            pltpu.VMEM((16, 16), jnp.float32),
            pltpu.VMEM((16, 16), jnp.float32),
            pltpu.SemaphoreType.DMA,
        ],
    )
    def combine(y_hbm, sh_hbm, route_hbm, w0b_hbm, w1b_hbm, out_hbm,
                y0_v, y1_v, sh_v, sf_v, idx0_v, idx1_v, w0_v, w1_v, sem):
        wid = lax.axis_index("s") * NC + lax.axis_index("c")
        base = wid * CH
        for ci in range(chunks):
            tb = base + ci * 16
            pltpu.sync_copy(route_hbm.at[0, pl.ds(tb, 16)], sf_v)
            idx0_v[...] = sf_v[...].astype(jnp.int32)
            pltpu.async_copy(y_hbm.at[idx0_v], y0_v, sem).wait()
            pltpu.sync_copy(route_hbm.at[1, pl.ds(tb, 16)], sf_v)
            idx1_v[...] = sf_v[...].astype(jnp.int32)
            pltpu.async_copy(y_hbm.at[idx1_v], y1_v, sem).wait()
            pltpu.sync_copy(w0b_hbm.at[pl.ds(tb, 16)], w0_v)
            pltpu.sync_copy(w1b_hbm.at[pl.ds(tb, 16)], w1_v)
            pltpu.sync_copy(sh_hbm.at[pl.ds(tb, 16)], sh_v)
            for r in range(16):
                w0r = w0_v[r]
                w1r = w1_v[r]

                def dbody(dd, _):
                    sl = pl.ds(dd * 16, 16)
                    sh_v[r, sl] = (sh_v[r, sl] + w0r * y0_v[r, sl]
                                   + w1r * y1_v[r, sl])
                    return 0

                lax.fori_loop(0, D // 16, dbody, 0)
            pltpu.sync_copy(sh_v, out_hbm.at[pl.ds(tb, 16)])

    return combine


def _dispatch_op(x, route):
    return _make_dispatch()(x, route)


def _combine_op(y, shared, route, w0b, w1b):
    return _make_combine()(y, shared, route, w0b, w1b)


def kernel(hidden_states, gate_weight, w_gate, w_up, w_down,
           sw_gate, sw_up, sw_down):
    b, s, d = hidden_states.shape
    ff = w_gate.shape[1]
    sff = sw_gate.shape[0]
    x = hidden_states.reshape(T, D)
    gwt = jnp.pad(gate_weight.T, ((0, 0), (0, EP - E)))

    route, cnt, xb, w0b, w1b = pl.pallas_call(
        _router_body,
        out_shape=(
            jax.ShapeDtypeStruct((EP, T), jnp.float32),
            jax.ShapeDtypeStruct((1, EP), jnp.float32),
            jax.ShapeDtypeStruct((T, D), jnp.bfloat16),
            jax.ShapeDtypeStruct((T, 16), jnp.float32),
            jax.ShapeDtypeStruct((T, 16), jnp.float32),
        ),
    )(x, gwt)

    cnt_i = cnt[0, :E].astype(jnp.int32)
    nblk = (jnp.maximum(cnt_i - C, 0) + OVB - 1) // OVB
    cumn = jnp.cumsum(nblk)
    na = cumn[-1]
    varange = jnp.arange(V2)
    vexp = jnp.minimum(
        jnp.sum((cumn[None, :] <= varange[:, None]).astype(jnp.int32), axis=1),
        E - 1)
    meta = jnp.concatenate([vexp, na[None]]).astype(jnp.int32)

    xd = _dispatch_op(x, route)

    tf = 512
    nf = ff // tf

    def xmap(g, f, m):
        nact = m[V2]
        return (jnp.where(g < E + nact, g, E + jnp.maximum(nact - 1, 0)), 0)

    def _wexpert(g, m):
        nact = m[V2]
        act = g < E + nact
        e_act = jnp.where(g < E, g, m[jnp.clip(g - E, 0, V2 - 1)])
        return jnp.where(act, e_act, m[jnp.maximum(nact - 1, 0)]), act

    def wmap(g, f, m):
        e, act = _wexpert(g, m)
        return (e, jnp.where(act, f, 0), 0)

    def wdmap(g, f, m):
        e, act = _wexpert(g, m)
        return (e, 0, jnp.where(act, f, 0))

    y = pl.pallas_call(
        _pass1_body,
        grid_spec=pltpu.PrefetchScalarGridSpec(
            num_scalar_prefetch=1,
            grid=(E + V2, nf),
            in_specs=[
                pl.BlockSpec((C, D), xmap),
                pl.BlockSpec((1, tf, D), wmap),
                pl.BlockSpec((1, tf, D), wmap),
                pl.BlockSpec((1, D, tf), wdmap),
            ],
            out_specs=pl.BlockSpec((C, D), xmap),
            scratch_shapes=[pltpu.VMEM((C, D), jnp.bfloat16)],
        ),
        out_shape=jax.ShapeDtypeStruct((R, D), jnp.float32),
    )(meta, xd, w_gate, w_up, w_down)

    tfs = 256
    nfs = sff // tfs
    shared = pl.pallas_call(
        _shared_body,
        grid=(nfs,),
        in_specs=[
            pl.BlockSpec((T, D), lambda fi: (0, 0)),
            pl.BlockSpec((tfs, D), lambda fi: (fi, 0)),
            pl.BlockSpec((tfs, D), lambda fi: (fi, 0)),
            pl.BlockSpec((D, tfs), lambda fi: (0, fi)),
        ],
        out_specs=pl.BlockSpec((T, D), lambda fi: (0, 0)),
        out_shape=jax.ShapeDtypeStruct((T, D), jnp.float32),
    )(xb, sw_gate, sw_up, sw_down)

    out = _combine_op(y, shared, route, w0b, w1b)
    return out.reshape(b, s, d)


# speedup vs baseline: 1.8692x; 1.0560x over previous
import functools

import jax
import jax.numpy as jnp
from jax import lax
from jax.experimental import pallas as pl
from jax.experimental.pallas import tpu as pltpu
from jax.experimental.pallas import tpu_sc as plsc

E = 8
EP = 128
T = 2048
D = 1024
C = 1024
OVB = 1024
V2 = 3
R = E * C + V2 * OVB
NC, NS = 2, 16
NW = NC * NS
CH = T // NW
TRI = 512


def _router_body(x_ref, gwt_ref, route_ref, cnt_ref, xb_ref,
                 w0b_ref, w1b_ref):
    x = x_ref[...]
    logits = lax.dot_general(
        x.astype(jnp.bfloat16), gwt_ref[...].astype(jnp.bfloat16),
        (((1,), (0,)), ((), ())), preferred_element_type=jnp.float32)
    col = lax.broadcasted_iota(jnp.int32, (T, EP), 1)
    logits = jnp.where(col < E, logits, jnp.float32(-1e30))
    m = jnp.max(logits, axis=1, keepdims=True)
    ex = jnp.exp(logits - m)
    scores = ex / jnp.sum(ex, axis=1, keepdims=True)
    s1 = jnp.max(scores, axis=1, keepdims=True)
    i1 = jnp.min(jnp.where(scores == s1, col, EP), axis=1, keepdims=True)
    oh1 = col == i1
    s2 = jnp.max(jnp.where(oh1, jnp.float32(-1.0), scores), axis=1,
                 keepdims=True)
    i2 = jnp.min(jnp.where((scores == s2) & (~oh1), col, EP), axis=1,
                 keepdims=True)
    oh2 = col == i2

    ri = lax.broadcasted_iota(jnp.int32, (TRI, TRI), 0)
    ci = lax.broadcasted_iota(jnp.int32, (TRI, TRI), 1)
    ltri = (ci < ri).astype(jnp.bfloat16)
    running = jnp.zeros((1, EP), jnp.float32)
    parts = []
    for oh in (oh1, oh2):
        ohf = oh.astype(jnp.float32)
        for blk in range(T // TRI):
            ohb = ohf[blk * TRI:(blk + 1) * TRI, :]
            within = lax.dot_general(
                ltri, ohb.astype(jnp.bfloat16), (((1,), (0,)), ((), ())),
                preferred_element_type=jnp.float32)
            ranks_blk = within + running
            parts.append(jnp.sum(ranks_blk * ohb, axis=1, keepdims=True))
            running = running + jnp.sum(ohb, axis=0, keepdims=True)
    r1 = jnp.concatenate(parts[:4], axis=0)
    r2 = jnp.concatenate(parts[4:], axis=0)
    cnt = running

    ri2 = lax.broadcasted_iota(jnp.int32, (EP, EP), 0)
    ci2 = lax.broadcasted_iota(jnp.int32, (EP, EP), 1)
    mlt = (ri2 < ci2).astype(jnp.float32)
    excl = functools.partial(
        lax.dot_general, dimension_numbers=(((1,), (0,)), ((), ())),
        precision=lax.Precision.HIGHEST, preferred_element_type=jnp.float32)
    off = excl(cnt, mlt)
    ov = jnp.maximum(cnt - C, 0.0)
    ovpad = jnp.floor((ov + (OVB - 1)) / OVB) * OVB
    p2o = excl(ovpad, mlt)

    ohf1 = oh1.astype(jnp.float32)
    ohf2 = oh2.astype(jnp.float32)
    def slot_for(ohf, idx, rk):
        base = jnp.sum(off * ohf, axis=1, keepdims=True)
        p2 = jnp.sum(p2o * ohf, axis=1, keepdims=True)
        ef = idx.astype(jnp.float32)
        return jnp.where(rk < C, ef * C + rk, E * C + p2 + (rk - C))
    slot0 = slot_for(ohf1, i1, r1)
    slot1 = slot_for(ohf2, i2, r2)
    w0 = s1
    w1 = s2

    g = (jnp.where(col == 0, slot0, 0.0) + jnp.where(col == 1, slot1, 0.0)
         + jnp.where(col == 2, w0, 0.0) + jnp.where(col == 3, w1, 0.0))
    route_ref[...] = lax.transpose(g, (1, 0))
    cnt_ref[...] = cnt
    xb_ref[...] = x.astype(jnp.bfloat16)
    w0b_ref[...] = jnp.broadcast_to(w0, (T, 16))
    w1b_ref[...] = jnp.broadcast_to(w1, (T, 16))


def _make_dispatch():
    mesh = plsc.VectorSubcoreMesh(core_axis_name="c", subcore_axis_name="s")

    @functools.partial(
        pl.kernel, mesh=mesh,
        out_type=jax.ShapeDtypeStruct((R, D), jnp.float32),
        scratch_types=[
            pltpu.VMEM((CH, D), jnp.float32),
            pltpu.VMEM((CH,), jnp.float32),
            pltpu.VMEM((CH,), jnp.int32),
            pltpu.SemaphoreType.DMA,
        ],
    )
    def dispatch(x_hbm, route_hbm, xd_hbm, rows_v, sf_v, idx_v, sem):
        wid = lax.axis_index("s") * NC + lax.axis_index("c")
        base = wid * CH
        pltpu.sync_copy(x_hbm.at[pl.ds(base, CH)], rows_v)
        for k in range(2):
            pltpu.sync_copy(route_hbm.at[k, pl.ds(base, CH)], sf_v)
            for c in range(CH // 16):
                idx_v[pl.ds(c * 16, 16)] = (
                    sf_v[pl.ds(c * 16, 16)].astype(jnp.int32))
            pltpu.async_copy(rows_v, xd_hbm.at[idx_v], sem).wait()

    return dispatch


def _pass1_body(m_ref, x_ref, wg_ref, wu_ref, wd_ref, y_ref, xb_scr):
    g = pl.program_id(0)
    f = pl.program_id(1)
    na = m_ref[V2]

    @pl.when(g < E + na)
    def _():
        @pl.when(f == 0)
        def _():
            xb_scr[...] = x_ref[...].astype(jnp.bfloat16)
            y_ref[...] = jnp.zeros_like(y_ref)

        xb = xb_scr[...]
        wg = wg_ref[0].astype(jnp.bfloat16)
        wu = wu_ref[0].astype(jnp.bfloat16)
        wd = wd_ref[0].astype(jnp.bfloat16)
        nt = (((1,), (1,)), ((), ()))
        a = lax.dot_general(xb, wg, nt, preferred_element_type=jnp.float32)
        bb = lax.dot_general(xb, wu, nt, preferred_element_type=jnp.float32)
        h = (a * jax.nn.sigmoid(a) * bb).astype(jnp.bfloat16)
        y_ref[...] += lax.dot_general(h, wd, nt,
                                      preferred_element_type=jnp.float32)


def _shared_body(xb_ref, wg_ref, wu_ref, wd_ref, out_ref):
    f = pl.program_id(0)

    @pl.when(f == 0)
    def _():
        out_ref[...] = jnp.zeros_like(out_ref)

    xb = xb_ref[...]
    wg = wg_ref[...].astype(jnp.bfloat16)
    wu = wu_ref[...].astype(jnp.bfloat16)
    wd = wd_ref[...].astype(jnp.bfloat16)
    nt = (((1,), (1,)), ((), ()))
    a = lax.dot_general(xb, wg, nt, preferred_element_type=jnp.float32)
    b = lax.dot_general(xb, wu, nt, preferred_element_type=jnp.float32)
    h = (a * jax.nn.sigmoid(a) * b).astype(jnp.bfloat16)
    out_ref[...] += lax.dot_general(h, wd, nt,
                                    preferred_element_type=jnp.float32)


def _make_combine():
    mesh = plsc.VectorSubcoreMesh(core_axis_name="c", subcore_axis_name="s")
    chunks = CH // 16

    nbuf = 2

    @functools.partial(
        pl.kernel, mesh=mesh,
        out_type=jax.ShapeDtypeStruct((T, D), jnp.float32),
        scratch_types=(
            [pltpu.VMEM((16, D), jnp.float32)] * (3 * nbuf)
            + [pltpu.VMEM((16,), jnp.float32)] * nbuf
            + [pltpu.VMEM((16,), jnp.int32)] * (2 * nbuf)
            + [pltpu.VMEM((16, 16), jnp.float32)] * (2 * nbuf)
            + [pltpu.SemaphoreType.DMA] * (3 * nbuf)
        ),
    )
    def combine(y_hbm, sh_hbm, route_hbm, w0b_hbm, w1b_hbm, out_hbm, *scr):
        y0b = scr[0:2]
        y1b = scr[2:4]
        shb = scr[4:6]
        sfb = scr[6:8]
        idx0b = scr[8:10]
        idx1b = scr[10:12]
        w0bv = scr[12:14]
        w1bv = scr[14:16]
        sems = scr[16:22]
        wid = lax.axis_index("s") * NC + lax.axis_index("c")
        base = wid * CH

        def issue(ci, bs):
            tb = base + ci * 16
            pltpu.sync_copy(route_hbm.at[0, pl.ds(tb, 16)], sfb[bs])
            idx0b[bs][...] = sfb[bs][...].astype(jnp.int32)
            pltpu.sync_copy(route_hbm.at[1, pl.ds(tb, 16)], sfb[bs])
            idx1b[bs][...] = sfb[bs][...].astype(jnp.int32)
            pltpu.sync_copy(w0b_hbm.at[pl.ds(tb, 16)], w0bv[bs])
            pltpu.sync_copy(w1b_hbm.at[pl.ds(tb, 16)], w1bv[bs])
            return (
                pltpu.async_copy(y_hbm.at[idx0b[bs]], y0b[bs], sems[3 * bs]),
                pltpu.async_copy(y_hbm.at[idx1b[bs]], y1b[bs],
                                 sems[3 * bs + 1]),
                pltpu.async_copy(sh_hbm.at[pl.ds(tb, 16)], shb[bs],
                                 sems[3 * bs + 2]),
            )

        def crunch(ci, bs, handles):
            for h in handles:
                h.wait()
            sh_v, y0_v, y1_v = shb[bs], y0b[bs], y1b[bs]
            w0_v, w1_v = w0bv[bs], w1bv[bs]

            def dbody(dd, _):
                sl = pl.ds(dd * 16, 16)
                for r in range(16):
                    sh_v[r, sl] = (sh_v[r, sl] + w0_v[r] * y0_v[r, sl]
                                   + w1_v[r] * y1_v[r, sl])
                return 0

            lax.fori_loop(0, D // 16, dbody, 0)
            pltpu.sync_copy(sh_v, out_hbm.at[pl.ds(base + ci * 16, 16)])

        pending = issue(0, 0)
        for ci in range(chunks):
            nxt = None
            if ci + 1 < chunks:
                nxt = issue(ci + 1, (ci + 1) % nbuf)
            crunch(ci, ci % nbuf, pending)
            pending = nxt

    return combine


def _dispatch_op(x, route):
    return _make_dispatch()(x, route)


def _combine_op(y, shared, route, w0b, w1b):
    return _make_combine()(y, shared, route, w0b, w1b)


def kernel(hidden_states, gate_weight, w_gate, w_up, w_down,
           sw_gate, sw_up, sw_down):
    b, s, d = hidden_states.shape
    ff = w_gate.shape[1]
    sff = sw_gate.shape[0]
    x = hidden_states.reshape(T, D)
    gwt = jnp.pad(gate_weight.T, ((0, 0), (0, EP - E)))

    route, cnt, xb, w0b, w1b = pl.pallas_call(
        _router_body,
        out_shape=(
            jax.ShapeDtypeStruct((EP, T), jnp.float32),
            jax.ShapeDtypeStruct((1, EP), jnp.float32),
            jax.ShapeDtypeStruct((T, D), jnp.bfloat16),
            jax.ShapeDtypeStruct((T, 16), jnp.float32),
            jax.ShapeDtypeStruct((T, 16), jnp.float32),
        ),
    )(x, gwt)

    cnt_i = cnt[0, :E].astype(jnp.int32)
    nblk = (jnp.maximum(cnt_i - C, 0) + OVB - 1) // OVB
    cumn = jnp.cumsum(nblk)
    na = cumn[-1]
    varange = jnp.arange(V2)
    vexp = jnp.minimum(
        jnp.sum((cumn[None, :] <= varange[:, None]).astype(jnp.int32), axis=1),
        E - 1)
    meta = jnp.concatenate([vexp, na[None]]).astype(jnp.int32)

    xd = _dispatch_op(x, route)

    tf = 512
    nf = ff // tf

    def xmap(g, f, m):
        nact = m[V2]
        return (jnp.where(g < E + nact, g, E + jnp.maximum(nact - 1, 0)), 0)

    def _wexpert(g, m):
        nact = m[V2]
        act = g < E + nact
        e_act = jnp.where(g < E, g, m[jnp.clip(g - E, 0, V2 - 1)])
        return jnp.where(act, e_act, m[jnp.maximum(nact - 1, 0)]), act

    def wmap(g, f, m):
        e, act = _wexpert(g, m)
        return (e, jnp.where(act, f, 0), 0)

    def wdmap(g, f, m):
        e, act = _wexpert(g, m)
        return (e, 0, jnp.where(act, f, 0))

    y = pl.pallas_call(
        _pass1_body,
        grid_spec=pltpu.PrefetchScalarGridSpec(
            num_scalar_prefetch=1,
            grid=(E + V2, nf),
            in_specs=[
                pl.BlockSpec((C, D), xmap),
                pl.BlockSpec((1, tf, D), wmap),
                pl.BlockSpec((1, tf, D), wmap),
                pl.BlockSpec((1, D, tf), wdmap),
            ],
            out_specs=pl.BlockSpec((C, D), xmap),
            scratch_shapes=[pltpu.VMEM((C, D), jnp.bfloat16)],
        ),
        out_shape=jax.ShapeDtypeStruct((R, D), jnp.float32),
    )(meta, xd, w_gate, w_up, w_down)

    tfs = 256
    nfs = sff // tfs
    shared = pl.pallas_call(
        _shared_body,
        grid=(nfs,),
        in_specs=[
            pl.BlockSpec((T, D), lambda fi: (0, 0)),
            pl.BlockSpec((tfs, D), lambda fi: (fi, 0)),
            pl.BlockSpec((tfs, D), lambda fi: (fi, 0)),
            pl.BlockSpec((D, tfs), lambda fi: (0, fi)),
        ],
        out_specs=pl.BlockSpec((T, D), lambda fi: (0, 0)),
        out_shape=jax.ShapeDtypeStruct((T, D), jnp.float32),
    )(xb, sw_gate, sw_up, sw_down)

    out = _combine_op(y, shared, route, w0b, w1b)
    return out.reshape(b, s, d)


# speedup vs baseline: 2.2644x; 1.2114x over previous
import functools

import jax
import jax.numpy as jnp
from jax import lax
from jax.experimental import pallas as pl
from jax.experimental.pallas import tpu as pltpu
from jax.experimental.pallas import tpu_sc as plsc

E = 8
EP = 128
T = 2048
D = 1024
C = 640
OVB = 640
V2 = 6
R = E * C + V2 * OVB
NC, NS = 2, 16
NW = NC * NS
CH = T // NW
TRI = 512


def _router_body(x_ref, gwt_ref, route_ref, cnt_ref, xb_ref,
                 w0b_ref, w1b_ref):
    x = x_ref[...]
    logits = lax.dot_general(
        x.astype(jnp.bfloat16), gwt_ref[...].astype(jnp.bfloat16),
        (((1,), (0,)), ((), ())), preferred_element_type=jnp.float32)
    col = lax.broadcasted_iota(jnp.int32, (T, EP), 1)
    logits = jnp.where(col < E, logits, jnp.float32(-1e30))
    m = jnp.max(logits, axis=1, keepdims=True)
    ex = jnp.exp(logits - m)
    scores = ex / jnp.sum(ex, axis=1, keepdims=True)
    s1 = jnp.max(scores, axis=1, keepdims=True)
    i1 = jnp.min(jnp.where(scores == s1, col, EP), axis=1, keepdims=True)
    oh1 = col == i1
    s2 = jnp.max(jnp.where(oh1, jnp.float32(-1.0), scores), axis=1,
                 keepdims=True)
    i2 = jnp.min(jnp.where((scores == s2) & (~oh1), col, EP), axis=1,
                 keepdims=True)
    oh2 = col == i2

    ri = lax.broadcasted_iota(jnp.int32, (TRI, TRI), 0)
    ci = lax.broadcasted_iota(jnp.int32, (TRI, TRI), 1)
    ltri = (ci < ri).astype(jnp.bfloat16)
    running = jnp.zeros((1, EP), jnp.float32)
    parts = []
    for oh in (oh1, oh2):
        ohf = oh.astype(jnp.float32)
        for blk in range(T // TRI):
            ohb = ohf[blk * TRI:(blk + 1) * TRI, :]
            within = lax.dot_general(
                ltri, ohb.astype(jnp.bfloat16), (((1,), (0,)), ((), ())),
                preferred_element_type=jnp.float32)
            ranks_blk = within + running
            parts.append(jnp.sum(ranks_blk * ohb, axis=1, keepdims=True))
            running = running + jnp.sum(ohb, axis=0, keepdims=True)
    r1 = jnp.concatenate(parts[:4], axis=0)
    r2 = jnp.concatenate(parts[4:], axis=0)
    cnt = running

    ri2 = lax.broadcasted_iota(jnp.int32, (EP, EP), 0)
    ci2 = lax.broadcasted_iota(jnp.int32, (EP, EP), 1)
    mlt = (ri2 < ci2).astype(jnp.float32)
    excl = functools.partial(
        lax.dot_general, dimension_numbers=(((1,), (0,)), ((), ())),
        precision=lax.Precision.HIGHEST, preferred_element_type=jnp.float32)
    off = excl(cnt, mlt)
    ov = jnp.maximum(cnt - C, 0.0)
    ovpad = jnp.floor((ov + (OVB - 1)) / OVB) * OVB
    p2o = excl(ovpad, mlt)

    ohf1 = oh1.astype(jnp.float32)
    ohf2 = oh2.astype(jnp.float32)
    def slot_for(ohf, idx, rk):
        base = jnp.sum(off * ohf, axis=1, keepdims=True)
        p2 = jnp.sum(p2o * ohf, axis=1, keepdims=True)
        ef = idx.astype(jnp.float32)
        return jnp.where(rk < C, ef * C + rk, E * C + p2 + (rk - C))
    slot0 = slot_for(ohf1, i1, r1)
    slot1 = slot_for(ohf2, i2, r2)
    w0 = s1
    w1 = s2

    g = (jnp.where(col == 0, slot0, 0.0) + jnp.where(col == 1, slot1, 0.0)
         + jnp.where(col == 2, w0, 0.0) + jnp.where(col == 3, w1, 0.0))
    route_ref[...] = lax.transpose(g, (1, 0))
    cnt_ref[...] = cnt
    xb_ref[...] = x.astype(jnp.bfloat16)
    w0b_ref[...] = jnp.broadcast_to(w0, (T, 16))
    w1b_ref[...] = jnp.broadcast_to(w1, (T, 16))


def _make_dispatch():
    mesh = plsc.VectorSubcoreMesh(core_axis_name="c", subcore_axis_name="s")

    @functools.partial(
        pl.kernel, mesh=mesh,
        out_type=jax.ShapeDtypeStruct((R, D), jnp.float32),
        scratch_types=[
            pltpu.VMEM((CH, D), jnp.float32),
            pltpu.VMEM((CH,), jnp.float32),
            pltpu.VMEM((CH,), jnp.int32),
            pltpu.VMEM((CH,), jnp.int32),
            pltpu.SemaphoreType.DMA,
            pltpu.SemaphoreType.DMA,
        ],
    )
    def dispatch(x_hbm, route_hbm, xd_hbm, rows_v, sf_v, idx0_v, idx1_v,
                 sem0, sem1):
        wid = lax.axis_index("s") * NC + lax.axis_index("c")
        base = wid * CH
        pltpu.sync_copy(x_hbm.at[pl.ds(base, CH)], rows_v)
        for k, (idx_v, sem) in enumerate(((idx0_v, sem0), (idx1_v, sem1))):
            pltpu.sync_copy(route_hbm.at[k, pl.ds(base, CH)], sf_v)
            for c in range(CH // 16):
                idx_v[pl.ds(c * 16, 16)] = (
                    sf_v[pl.ds(c * 16, 16)].astype(jnp.int32))
        h0 = pltpu.async_copy(rows_v, xd_hbm.at[idx0_v], sem0)
        h1 = pltpu.async_copy(rows_v, xd_hbm.at[idx1_v], sem1)
        h0.wait()
        h1.wait()

    return dispatch


def _pass1_body(m_ref, x_ref, wg_ref, wu_ref, wd_ref, y_ref, xb_scr):
    g = pl.program_id(0)
    f = pl.program_id(1)
    na = m_ref[V2]

    @pl.when(g < E + na)
    def _():
        @pl.when(f == 0)
        def _():
            xb_scr[...] = x_ref[...].astype(jnp.bfloat16)
            y_ref[...] = jnp.zeros_like(y_ref)

        xb = xb_scr[...]
        wg = wg_ref[0].astype(jnp.bfloat16)
        wu = wu_ref[0].astype(jnp.bfloat16)
        wd = wd_ref[0].astype(jnp.bfloat16)
        nt = (((1,), (1,)), ((), ()))
        a = lax.dot_general(xb, wg, nt, preferred_element_type=jnp.float32)
        bb = lax.dot_general(xb, wu, nt, preferred_element_type=jnp.float32)
        h = (a * jax.nn.sigmoid(a) * bb).astype(jnp.bfloat16)
        y_ref[...] += lax.dot_general(h, wd, nt,
                                      preferred_element_type=jnp.float32)


def _shared_body(xb_ref, wg_ref, wu_ref, wd_ref, out_ref):
    f = pl.program_id(0)

    @pl.when(f == 0)
    def _():
        out_ref[...] = jnp.zeros_like(out_ref)

    xb = xb_ref[...]
    wg = wg_ref[...].astype(jnp.bfloat16)
    wu = wu_ref[...].astype(jnp.bfloat16)
    wd = wd_ref[...].astype(jnp.bfloat16)
    nt = (((1,), (1,)), ((), ()))
    a = lax.dot_general(xb, wg, nt, preferred_element_type=jnp.float32)
    b = lax.dot_general(xb, wu, nt, preferred_element_type=jnp.float32)
    h = (a * jax.nn.sigmoid(a) * b).astype(jnp.bfloat16)
    out_ref[...] += lax.dot_general(h, wd, nt,
                                    preferred_element_type=jnp.float32)


def _make_combine():
    mesh = plsc.VectorSubcoreMesh(core_axis_name="c", subcore_axis_name="s")
    chunks = CH // 16

    nbuf = 2

    @functools.partial(
        pl.kernel, mesh=mesh,
        out_type=jax.ShapeDtypeStruct((T, D), jnp.float32),
        scratch_types=(
            [pltpu.VMEM((16, D), jnp.float32)] * (3 * nbuf)
            + [pltpu.VMEM((16,), jnp.float32)] * nbuf
            + [pltpu.VMEM((16,), jnp.int32)] * (2 * nbuf)
            + [pltpu.VMEM((16, 16), jnp.float32)] * (2 * nbuf)
            + [pltpu.SemaphoreType.DMA] * (3 * nbuf)
        ),
    )
    def combine(y_hbm, sh_hbm, route_hbm, w0b_hbm, w1b_hbm, out_hbm, *scr):
        y0b = scr[0:2]
        y1b = scr[2:4]
        shb = scr[4:6]
        sfb = scr[6:8]
        idx0b = scr[8:10]
        idx1b = scr[10:12]
        w0bv = scr[12:14]
        w1bv = scr[14:16]
        sems = scr[16:22]
        wid = lax.axis_index("s") * NC + lax.axis_index("c")
        base = wid * CH

        def issue(ci, bs):
            tb = base + ci * 16
            pltpu.sync_copy(route_hbm.at[0, pl.ds(tb, 16)], sfb[bs])
            idx0b[bs][...] = sfb[bs][...].astype(jnp.int32)
            pltpu.sync_copy(route_hbm.at[1, pl.ds(tb, 16)], sfb[bs])
            idx1b[bs][...] = sfb[bs][...].astype(jnp.int32)
            pltpu.sync_copy(w0b_hbm.at[pl.ds(tb, 16)], w0bv[bs])
            pltpu.sync_copy(w1b_hbm.at[pl.ds(tb, 16)], w1bv[bs])
            return (
                pltpu.async_copy(y_hbm.at[idx0b[bs]], y0b[bs], sems[3 * bs]),
                pltpu.async_copy(y_hbm.at[idx1b[bs]], y1b[bs],
                                 sems[3 * bs + 1]),
                pltpu.async_copy(sh_hbm.at[pl.ds(tb, 16)], shb[bs],
                                 sems[3 * bs + 2]),
            )

        def crunch(ci, bs, handles):
            for h in handles:
                h.wait()
            sh_v, y0_v, y1_v = shb[bs], y0b[bs], y1b[bs]
            w0_v, w1_v = w0bv[bs], w1bv[bs]

            def dbody(dd, _):
                sl = pl.ds(dd * 16, 16)
                for r in range(16):
                    sh_v[r, sl] = (sh_v[r, sl] + w0_v[r] * y0_v[r, sl]
                                   + w1_v[r] * y1_v[r, sl])
                return 0

            lax.fori_loop(0, D // 16, dbody, 0)
            pltpu.sync_copy(sh_v, out_hbm.at[pl.ds(base + ci * 16, 16)])

        pending = issue(0, 0)
        for ci in range(chunks):
            nxt = None
            if ci + 1 < chunks:
                nxt = issue(ci + 1, (ci + 1) % nbuf)
            crunch(ci, ci % nbuf, pending)
            pending = nxt

    return combine


def _dispatch_op(x, route):
    return _make_dispatch()(x, route)


def _combine_op(y, shared, route, w0b, w1b):
    return _make_combine()(y, shared, route, w0b, w1b)


def kernel(hidden_states, gate_weight, w_gate, w_up, w_down,
           sw_gate, sw_up, sw_down):
    b, s, d = hidden_states.shape
    ff = w_gate.shape[1]
    sff = sw_gate.shape[0]
    x = hidden_states.reshape(T, D)
    gwt = jnp.pad(gate_weight.T, ((0, 0), (0, EP - E)))

    route, cnt, xb, w0b, w1b = pl.pallas_call(
        _router_body,
        out_shape=(
            jax.ShapeDtypeStruct((EP, T), jnp.float32),
            jax.ShapeDtypeStruct((1, EP), jnp.float32),
            jax.ShapeDtypeStruct((T, D), jnp.bfloat16),
            jax.ShapeDtypeStruct((T, 16), jnp.float32),
            jax.ShapeDtypeStruct((T, 16), jnp.float32),
        ),
    )(x, gwt)

    cnt_i = cnt[0, :E].astype(jnp.int32)
    nblk = (jnp.maximum(cnt_i - C, 0) + OVB - 1) // OVB
    cumn = jnp.cumsum(nblk)
    na = cumn[-1]
    varange = jnp.arange(V2)
    vexp = jnp.minimum(
        jnp.sum((cumn[None, :] <= varange[:, None]).astype(jnp.int32), axis=1),
        E - 1)
    meta = jnp.concatenate([vexp, na[None]]).astype(jnp.int32)

    xd = _dispatch_op(x, route)

    tf = 512
    nf = ff // tf

    def xmap(g, f, m):
        nact = m[V2]
        return (jnp.where(g < E + nact, g, E + jnp.maximum(nact - 1, 0)), 0)

    def _wexpert(g, m):
        nact = m[V2]
        act = g < E + nact
        e_act = jnp.where(g < E, g, m[jnp.clip(g - E, 0, V2 - 1)])
        return jnp.where(act, e_act, m[jnp.maximum(nact - 1, 0)]), act

    def wmap(g, f, m):
        e, act = _wexpert(g, m)
        return (e, jnp.where(act, f, 0), 0)

    def wdmap(g, f, m):
        e, act = _wexpert(g, m)
        return (e, 0, jnp.where(act, f, 0))

    y = pl.pallas_call(
        _pass1_body,
        grid_spec=pltpu.PrefetchScalarGridSpec(
            num_scalar_prefetch=1,
            grid=(E + V2, nf),
            in_specs=[
                pl.BlockSpec((C, D), xmap),
                pl.BlockSpec((1, tf, D), wmap),
                pl.BlockSpec((1, tf, D), wmap),
                pl.BlockSpec((1, D, tf), wdmap),
            ],
            out_specs=pl.BlockSpec((C, D), xmap),
            scratch_shapes=[pltpu.VMEM((C, D), jnp.bfloat16)],
        ),
        out_shape=jax.ShapeDtypeStruct((R, D), jnp.float32),
    )(meta, xd, w_gate, w_up, w_down)

    tfs = 256
    nfs = sff // tfs
    shared = pl.pallas_call(
        _shared_body,
        grid=(nfs,),
        in_specs=[
            pl.BlockSpec((T, D), lambda fi: (0, 0)),
            pl.BlockSpec((tfs, D), lambda fi: (fi, 0)),
            pl.BlockSpec((tfs, D), lambda fi: (fi, 0)),
            pl.BlockSpec((D, tfs), lambda fi: (0, fi)),
        ],
        out_specs=pl.BlockSpec((T, D), lambda fi: (0, 0)),
        out_shape=jax.ShapeDtypeStruct((T, D), jnp.float32),
    )(xb, sw_gate, sw_up, sw_down)

    out = _combine_op(y, shared, route, w0b, w1b)
    return out.reshape(b, s, d)
